# baseline (speedup 1.0000x reference)
import functools

import jax
import jax.numpy as jnp
from jax import lax
from jax.experimental import pallas as pl
from jax.experimental.pallas import tpu as pltpu
from jax.experimental.pallas import tpu_sc as plsc

NC, NS, L = 2, 16, 16
NW = NC * NS

ROWS, COLS = 16384, 1024
ROWS_W = ROWS // NW
CR = 8
NBUF = 4
NCHUNK = ROWS_W // CR
VEC_PER_CHUNK = CR * COLS // L

_MAGIC = jnp.float32(12582912.0)


def _sc_body(data_hbm, tbl_hbm, scale_hbm, out_hbm,
             tbl_v, scale_v, lut_v, inb, outb, sin, sout):
    wid = lax.axis_index("s") * NC + lax.axis_index("c")
    base = wid * ROWS_W

    for b in range(NBUF):
        pltpu.async_copy(
            data_hbm.at[pl.ds(base + b * CR, CR)], inb[b], sin[b]
        )

    pltpu.sync_copy(tbl_hbm, tbl_v)
    pltpu.sync_copy(scale_hbm, scale_v)
    scale = plsc.load_gather(scale_v, [jnp.zeros((L,), jnp.int32)])

    @pl.loop(0, 256 // L)
    def _build(k):
        y = tbl_v[pl.ds(k * L, L)] * 128.0
        y = (y + _MAGIC) - _MAGIC
        y = jnp.minimum(jnp.maximum(y, -128.0), 127.0)
        lut_v[pl.ds(k * L, L)] = y * scale

    @pl.loop(0, NCHUNK, step=NBUF)
    def _group(g0):
        for b in range(NBUF):
            g = g0 + b
            row = base + g * CR

            pltpu.make_async_copy(
                data_hbm.at[pl.ds(row, CR)], inb[b], sin[b]
            ).wait()

            @pl.when(g0 >= NBUF)
            def _():
                pltpu.make_async_copy(
                    outb[b], out_hbm.at[pl.ds(row - NBUF * CR, CR)], sout[b]
                ).wait()

            @plsc.parallel_loop(0, VEC_PER_CHUNK, unroll=8)
            def _vec(i):
                r = i >> 6
                c = (i & 63) << 4
                d = inb[b][r, pl.ds(c, L)]
                outb[b][r, pl.ds(c, L)] = plsc.load_gather(lut_v, [d + 128])

            pltpu.async_copy(outb[b], out_hbm.at[pl.ds(row, CR)], sout[b])

            @pl.when(g0 + NBUF < NCHUNK)
            def _():
                pltpu.async_copy(
                    data_hbm.at[pl.ds(row + NBUF * CR, CR)], inb[b], sin[b]
                )

    for b in range(NBUF):
        row = base + (NCHUNK - NBUF + b) * CR
        pltpu.make_async_copy(
            outb[b], out_hbm.at[pl.ds(row, CR)], sout[b]
        ).wait()


@functools.partial(
    pl.kernel,
    out_type=jax.ShapeDtypeStruct((ROWS, COLS), jnp.float32),
    mesh=plsc.VectorSubcoreMesh(
        core_axis_name="c", subcore_axis_name="s", num_cores=NC, num_subcores=NS
    ),
    scratch_types=(
        [
            pltpu.VMEM((256,), jnp.float32),
            pltpu.VMEM((1,), jnp.float32),
            pltpu.VMEM((256,), jnp.float32),
        ]
        + [pltpu.VMEM((CR, COLS), jnp.int32) for _ in range(NBUF)]
        + [pltpu.VMEM((CR, COLS), jnp.float32) for _ in range(NBUF)]
        + [pltpu.SemaphoreType.DMA for _ in range(2 * NBUF)]
    ),
    compiler_params=pltpu.CompilerParams(needs_layout_passes=False),
)
def _sc_lookup(data_hbm, tbl_hbm, scale_hbm, out_hbm, tbl_v, scale_v, lut_v, *bufs):
    inb = list(bufs[:NBUF])
    outb = list(bufs[NBUF:2 * NBUF])
    sin = list(bufs[2 * NBUF:3 * NBUF])
    sout = list(bufs[3 * NBUF:4 * NBUF])
    _sc_body(data_hbm, tbl_hbm, scale_hbm, out_hbm,
             tbl_v, scale_v, lut_v, inb, outb, sin, sout)


@jax.jit
def kernel(data, float_table, out_scale):
    return _sc_lookup(data, float_table, out_scale)


# speedup vs baseline: 1.0247x; 1.0247x over previous
import functools

import jax
import jax.numpy as jnp
from jax import lax
from jax.experimental import pallas as pl
from jax.experimental.pallas import tpu as pltpu
from jax.experimental.pallas import tpu_sc as plsc

NC, NS, L = 2, 16, 16
NW = NC * NS

ROWS, COLS = 16384, 1024
ROWS_W = ROWS // NW
CR = 8
NBUF = 4
NCHUNK = ROWS_W // CR
VEC_PER_CHUNK = CR * COLS // L


def _sc_body(data_hbm, lut_hbm, out_hbm, lut_v, inb, outb, sin, sout):
    wid = lax.axis_index("s") * NC + lax.axis_index("c")
    base = wid * ROWS_W

    for b in range(NBUF):
        pltpu.async_copy(
            data_hbm.at[pl.ds(base + b * CR, CR)], inb[b], sin[b]
        )

    pltpu.sync_copy(lut_hbm, lut_v)

    @pl.loop(0, NCHUNK, step=NBUF)
    def _group(g0):
        for b in range(NBUF):
            g = g0 + b
            row = base + g * CR

            pltpu.make_async_copy(
                data_hbm.at[pl.ds(row, CR)], inb[b], sin[b]
            ).wait()

            @pl.when(g0 >= NBUF)
            def _():
                pltpu.make_async_copy(
                    outb[b], out_hbm.at[pl.ds(row - NBUF * CR, CR)], sout[b]
                ).wait()

            @plsc.parallel_loop(0, VEC_PER_CHUNK, unroll=8)
            def _vec(i):
                r = i >> 6
                c = (i & 63) << 4
                d = inb[b][r, pl.ds(c, L)]
                outb[b][r, pl.ds(c, L)] = plsc.load_gather(lut_v, [d + 128])

            pltpu.async_copy(outb[b], out_hbm.at[pl.ds(row, CR)], sout[b])

            @pl.when(g0 + NBUF < NCHUNK)
            def _():
                pltpu.async_copy(
                    data_hbm.at[pl.ds(row + NBUF * CR, CR)], inb[b], sin[b]
                )

    for b in range(NBUF):
        row = base + (NCHUNK - NBUF + b) * CR
        pltpu.make_async_copy(
            outb[b], out_hbm.at[pl.ds(row, CR)], sout[b]
        ).wait()


@functools.partial(
    pl.kernel,
    out_type=jax.ShapeDtypeStruct((ROWS, COLS), jnp.float32),
    mesh=plsc.VectorSubcoreMesh(
        core_axis_name="c", subcore_axis_name="s", num_cores=NC, num_subcores=NS
    ),
    scratch_types=(
        [pltpu.VMEM((256,), jnp.float32)]
        + [pltpu.VMEM((CR, COLS), jnp.int32) for _ in range(NBUF)]
        + [pltpu.VMEM((CR, COLS), jnp.float32) for _ in range(NBUF)]
        + [pltpu.SemaphoreType.DMA for _ in range(2 * NBUF)]
    ),
    compiler_params=pltpu.CompilerParams(needs_layout_passes=False),
)
def _sc_lookup(data_hbm, lut_hbm, out_hbm, lut_v, *bufs):
    inb = list(bufs[:NBUF])
    outb = list(bufs[NBUF:2 * NBUF])
    sin = list(bufs[2 * NBUF:3 * NBUF])
    sout = list(bufs[3 * NBUF:4 * NBUF])
    _sc_body(data_hbm, lut_hbm, out_hbm, lut_v, inb, outb, sin, sout)


@jax.jit
def kernel(data, float_table, out_scale):
    table_int = jnp.round(float_table * 128.0).astype(jnp.int32)
    table_int = jnp.clip(table_int, -128, 127)
    lut = table_int.astype(jnp.float32) * out_scale[0]
    return _sc_lookup(data, lut)
